# Initial kernel scaffold; baseline (speedup 1.0000x reference)
import jax
import jax.numpy as jnp
from jax.experimental import pallas as pl


def _finalize_body(acc_ref, s_ref, b_ref, o_ref):
    o_ref[...] = acc_ref[...] / (s_ref[...] + 1e-16) + b_ref[...]


def _gat_accum(x_src, x_dst, src, dst, ew, W_s, W_d, a_s, a_d, n_dst):
    h_s = x_src @ W_s
    h_d = x_dst @ W_d
    al_s = (h_s * a_s).sum(axis=-1)
    al_d = (h_d * a_d).sum(axis=-1)
    e = al_s[src] + al_d[dst]
    e = jax.nn.leaky_relu(e, negative_slope=0.2)
    p = jnp.exp(e)
    s = jax.ops.segment_sum(p, dst, num_segments=n_dst)
    msg = (p * ew)[:, None] * h_s[src]
    acc = jax.ops.segment_sum(msg, dst, num_segments=n_dst)
    return acc, s


def kernel(x_user, x_badge, edge_index, edge_weight,
           W1_src, W1_dst, att1_src, att1_dst, b1,
           W2_src, W2_dst, att2_src, att2_dst, b2):
    src = edge_index[0]
    dst = edge_index[1]
    n = x_badge.shape[0]

    acc1, s1 = _gat_accum(x_user, x_badge, src, dst, edge_weight,
                          W1_src, W1_dst, att1_src, att1_dst, n)
    H1 = W1_src.shape[1]
    badge1 = pl.pallas_call(
        _finalize_body,
        out_shape=jax.ShapeDtypeStruct((n, H1), jnp.float32),
    )(acc1, s1[:, None], jnp.broadcast_to(b1[None, :], (n, H1)))
    badge1 = jax.nn.relu(badge1)

    user1 = jax.nn.relu(x_user @ W1_src)
    acc2, s2 = _gat_accum(user1, badge1, src, dst, edge_weight,
                          W2_src, W2_dst, att2_src, att2_dst, n)
    H2 = W2_src.shape[1]
    out = pl.pallas_call(
        _finalize_body,
        out_shape=jax.ShapeDtypeStruct((n, H2), jnp.float32),
    )(acc2, s2[:, None], jnp.broadcast_to(b2[None, :], (n, H2)))
    return out


# speedup vs baseline: 1.7092x; 1.7092x over previous
import jax
import jax.numpy as jnp
from jax.experimental import pallas as pl


def _finalize_body(acc_ref, s_ref, b_ref, o_ref):
    o_ref[...] = acc_ref[...] / (s_ref[...] + 1e-16) + b_ref[...]


def _gat_accum(x_src, x_dst, src, dst, ew, W_s, W_d, a_s, a_d, n_dst):
    h_s = x_src @ W_s
    h_d = x_dst @ W_d
    al_s = (h_s * a_s).sum(axis=-1)
    al_d = (h_d * a_d).sum(axis=-1)
    e = al_s[src] + al_d[dst]
    e = jax.nn.leaky_relu(e, negative_slope=0.2)
    p = jnp.exp(e)
    s = jax.ops.segment_sum(p, dst, num_segments=n_dst)
    msg = (p * ew)[:, None] * h_s[src]
    acc = jax.ops.segment_sum(msg, dst, num_segments=n_dst)
    return acc, s


def kernel(x_user, x_badge, edge_index, edge_weight,
           W1_src, W1_dst, att1_src, att1_dst, b1,
           W2_src, W2_dst, att2_src, att2_dst, b2):
    src = edge_index[0]
    dst = edge_index[1]
    n = x_badge.shape[0]

    acc1, s1 = _gat_accum(x_user, x_badge, src, dst, edge_weight,
                          W1_src, W1_dst, att1_src, att1_dst, n)
    H1 = W1_src.shape[1]
    BR = 10000
    badge1 = pl.pallas_call(
        _finalize_body,
        grid=(n // BR,),
        in_specs=[
            pl.BlockSpec((BR, H1), lambda i: (i, 0)),
            pl.BlockSpec((BR, 1), lambda i: (i, 0)),
            pl.BlockSpec((1, H1), lambda i: (0, 0)),
        ],
        out_specs=pl.BlockSpec((BR, H1), lambda i: (i, 0)),
        out_shape=jax.ShapeDtypeStruct((n, H1), jnp.float32),
    )(acc1, s1[:, None], b1[None, :])
    badge1 = jax.nn.relu(badge1)

    user1 = jax.nn.relu(x_user @ W1_src)
    acc2, s2 = _gat_accum(user1, badge1, src, dst, edge_weight,
                          W2_src, W2_dst, att2_src, att2_dst, n)
    H2 = W2_src.shape[1]
    out = pl.pallas_call(
        _finalize_body,
        grid=(n // BR,),
        in_specs=[
            pl.BlockSpec((BR, H2), lambda i: (i, 0)),
            pl.BlockSpec((BR, 1), lambda i: (i, 0)),
            pl.BlockSpec((1, H2), lambda i: (0, 0)),
        ],
        out_specs=pl.BlockSpec((BR, H2), lambda i: (i, 0)),
        out_shape=jax.ShapeDtypeStruct((n, H2), jnp.float32),
    )(acc2, s2[:, None], b2[None, :])
    return out


# speedup vs baseline: 65.3177x; 38.2142x over previous
import functools

import jax
import jax.numpy as jnp
from jax import lax
from jax.experimental import pallas as pl
from jax.experimental.pallas import tpu as pltpu
from jax.experimental.pallas import tpu_sc as plsc

N = 100000
NH = N // 2
E = 1600000
NS = 16
L = 16

K = 400
SB = 2000
EPS = 1e-16
CHUNKS = E // NS // K
STAGE = 10000
STAGE_H = 5000

_mesh = plsc.VectorSubcoreMesh(core_axis_name="c", subcore_axis_name="s")
_SC_PARAMS = pltpu.CompilerParams(needs_layout_passes=False,
                                  use_tc_tiling_on_sc=False)


def _sc_layer1(src_h, dst_h, ew_h, h_hbm, als_h, ald_h,
               acc_o, s_o,
               acc_sp, s_sp,
               src_a, dst_a, ew_a, als_a, ald_a, hrow_a, msg_a, p_a, dl_a,
               src_b, dst_b, ew_b, als_b, ald_b, hrow_b, msg_b, p_b, dl_b,
               c_b, sb,
               sem_e, sem_g, sem_s):
    cid = lax.axis_index("c")
    sid = lax.axis_index("s")
    is_hi = cid == 1

    A = (src_a, dst_a, ew_a, als_a, ald_a, hrow_a, msg_a, p_a, dl_a)
    B = (src_b, dst_b, ew_b, als_b, ald_b, hrow_b, msg_b, p_b, dl_b)

    zv = jnp.zeros((L,), jnp.float32)

    def _zm(j, _):
        msg_a[j] = zv
        return ()

    lax.fori_loop(0, K, _zm, (), unroll=8)

    def _zs(j, _):
        sb[pl.ds(j * L, L)] = zv
        return ()

    lax.fori_loop(0, SB // L, _zs, (), unroll=8)

    @pl.when(sid < 10)
    def _zero_acc():
        for t in range(STAGE_H // 200):
            rs = pl.ds(sid * STAGE_H + t * 200, 200)
            pltpu.sync_copy(msg_a.at[pl.ds(0, 200)], acc_sp.at[rs])

    @pl.when(sid < 5)
    def _zero_s():
        for t in range(STAGE // SB):
            rs = pl.ds(sid * STAGE + t * SB, SB)
            pltpu.sync_copy(sb, s_sp.at[rs])

    plsc.subcore_barrier()

    ebase = sid * (E // NS)

    def _issue_eb(bufs, k):
        es = pl.ds(ebase + k * K, K)
        pltpu.async_copy(src_h.at[es], bufs[0], sem_e)
        pltpu.async_copy(dst_h.at[es], bufs[1], sem_e)
        pltpu.async_copy(ew_h.at[es], bufs[2], sem_e)

    def _wait_eb(bufs, k):
        es = pl.ds(ebase + k * K, K)
        pltpu.make_async_copy(src_h.at[es], bufs[0], sem_e).wait()
        pltpu.make_async_copy(dst_h.at[es], bufs[1], sem_e).wait()
        pltpu.make_async_copy(ew_h.at[es], bufs[2], sem_e).wait()

    def _issue_g(bufs):
        pltpu.async_copy(als_h.at[bufs[0]], bufs[3], sem_g)
        pltpu.async_copy(ald_h.at[bufs[1]], bufs[4], sem_g)
        pltpu.async_copy(h_hbm.at[bufs[0]], bufs[5], sem_g)

    def _wait_g(bufs):
        pltpu.make_async_copy(als_h.at[bufs[0]], bufs[3], sem_g).wait()
        pltpu.make_async_copy(ald_h.at[bufs[1]], bufs[4], sem_g).wait()
        pltpu.make_async_copy(h_hbm.at[bufs[0]], bufs[5], sem_g).wait()

    def _issue_sc(bufs):
        pltpu.async_copy(bufs[6], acc_sp.at[bufs[8]], sem_s, add=True)
        pltpu.async_copy(bufs[7], s_sp.at[bufs[8]], sem_s, add=True)

    def _wait_sc(bufs):
        pltpu.make_async_copy(bufs[6], acc_sp.at[bufs[8]], sem_s).wait()
        pltpu.make_async_copy(bufs[7], s_sp.at[bufs[8]], sem_s).wait()

    def _vec(bufs):
        (_, dstr, ewr, alsr, aldr, _, _, pr, dlr) = bufs

        def vec(g, _):
            sl = pl.ds(g * L, L)
            e = alsr[sl] + aldr[sl]
            e = jnp.maximum(e, 0.2 * e)
            p = jnp.exp(e)
            dv = dstr[sl]
            ge = dv >= NH
            mine = ge == is_hi
            zero = jnp.zeros((L,), jnp.float32)
            c_b[sl] = jnp.where(mine, p * ewr[sl], zero)
            pr[sl] = jnp.where(mine, p, zero)
            dlr[sl] = jnp.where(ge, dv - NH, dv)
            return ()

        lax.fori_loop(0, K // L, vec, (), unroll=4)

    def _mrow(bufs):
        hrowr = bufs[5]
        msgr = bufs[6]

        def mrow(g, _):
            base = g * L
            c16 = c_b[pl.ds(base, L)]
            for l in range(L):
                msgr[base + l] = hrowr[base + l] * c16[l]
            return ()

        lax.fori_loop(0, K // L, mrow, (), unroll=1)

    _issue_eb(A, 0)

    def body2(i, _):
        a = 2 * i
        b = a + 1
        _wait_eb(A, a)
        _issue_g(A)

        @pl.when(i > 0)
        def _():
            _wait_g(B)
            _vec(B)
            _issue_eb(B, b)
            _mrow(B)
            _wait_sc(A)
            _issue_sc(B)

        @pl.when(i == 0)
        def _():
            _issue_eb(B, b)

        _wait_eb(B, b)
        _issue_g(B)
        _wait_g(A)
        _vec(A)

        @pl.when(i < CHUNKS // 2 - 1)
        def _():
            _issue_eb(A, a + 2)

        _mrow(A)

        @pl.when(i > 0)
        def _():
            _wait_sc(B)
        _issue_sc(A)
        return ()

    lax.fori_loop(0, CHUNKS // 2, body2, ())
    _wait_g(B)
    _vec(B)
    _mrow(B)
    _wait_sc(A)
    _issue_sc(B)
    _wait_sc(B)

    plsc.subcore_barrier()

    @pl.when(sid < 10)
    def _export():
        for t in range(STAGE_H // 200):
            loc = sid * STAGE_H + t * 200
            rs = pl.ds(loc, 200)
            rg = pl.ds(cid * NH + loc, 200)
            pltpu.sync_copy(acc_sp.at[rs], msg_a.at[pl.ds(0, 200)])
            pltpu.sync_copy(msg_a.at[pl.ds(0, 200)], acc_o.at[rg])

    @pl.when(sid < 5)
    def _export_s():
        for t in range(STAGE // SB):
            loc = sid * STAGE + t * SB
            rs = pl.ds(loc, SB)
            rg = pl.ds(cid * NH + loc, SB)
            pltpu.sync_copy(s_sp.at[rs], sb)
            pltpu.sync_copy(sb, s_o.at[rg])


def _k_bufs():
    return [
        pltpu.VMEM((K,), jnp.int32),
        pltpu.VMEM((K,), jnp.int32),
        pltpu.VMEM((K,), jnp.float32),
        pltpu.VMEM((K,), jnp.float32),
        pltpu.VMEM((K,), jnp.float32),
        pltpu.VMEM((K, 16), jnp.float32),
        pltpu.VMEM((K, 16), jnp.float32),
        pltpu.VMEM((K,), jnp.float32),
        pltpu.VMEM((K,), jnp.int32),
    ]


_layer1_call = pl.kernel(
    _sc_layer1,
    out_type=(
        jax.ShapeDtypeStruct((N, 16), jnp.float32),
        jax.ShapeDtypeStruct((N,), jnp.float32),
    ),
    mesh=_mesh,
    compiler_params=_SC_PARAMS,
    scratch_types=[
        pltpu.VMEM_SHARED((NH, 16), jnp.float32),
        pltpu.VMEM_SHARED((NH,), jnp.float32),
        *_k_bufs(),
        *_k_bufs(),
        pltpu.VMEM((K,), jnp.float32),
        pltpu.VMEM((SB,), jnp.float32),
        pltpu.SemaphoreType.DMA,
        pltpu.SemaphoreType.DMA,
        pltpu.SemaphoreType.DMA,
    ],
)


def _sc_layer2(src_h, dst_h, ew_h, rec_h, ald2_h,
               acc0_o, acc1_o, s_o,
               acc_sp, s_sp,
               src_a, dst_a, ew_a, rec_a, ald_a, m_a, p_a, dl_a,
               src_b, dst_b, ew_b, rec_b, ald_b, m_b, p_b, dl_b,
               sb,
               sem_e, sem_g, sem_s):
    cid = lax.axis_index("c")
    sid = lax.axis_index("s")

    A = (src_a, dst_a, ew_a, rec_a, ald_a, m_a, p_a, dl_a)
    B = (src_b, dst_b, ew_b, rec_b, ald_b, m_b, p_b, dl_b)

    zv = jnp.zeros((L,), jnp.float32)

    def _zs(j, _):
        sb[pl.ds(j * L, L)] = zv
        return ()

    lax.fori_loop(0, SB // L, _zs, (), unroll=8)

    @pl.when(sid < 10)
    def _zero():
        for t in range(STAGE // SB):
            rs = pl.ds(sid * STAGE + t * SB, SB)
            pltpu.sync_copy(sb, acc_sp.at[rs])
            pltpu.sync_copy(sb, s_sp.at[rs])

    plsc.subcore_barrier()

    ebase = sid * (E // NS)
    iota = lax.iota(jnp.int32, L)
    cidv = jnp.full((L,), cid, jnp.int32)
    col2 = jnp.full((L,), 2, jnp.int32)

    def _issue_eb(bufs, k):
        es = pl.ds(ebase + k * K, K)
        pltpu.async_copy(src_h.at[es], bufs[0], sem_e)
        pltpu.async_copy(dst_h.at[es], bufs[1], sem_e)
        pltpu.async_copy(ew_h.at[es], bufs[2], sem_e)

    def _wait_eb(bufs, k):
        es = pl.ds(ebase + k * K, K)
        pltpu.make_async_copy(src_h.at[es], bufs[0], sem_e).wait()
        pltpu.make_async_copy(dst_h.at[es], bufs[1], sem_e).wait()
        pltpu.make_async_copy(ew_h.at[es], bufs[2], sem_e).wait()

    def _issue_g(bufs):
        pltpu.async_copy(rec_h.at[bufs[0]], bufs[3], sem_g)
        pltpu.async_copy(ald2_h.at[bufs[1]], bufs[4], sem_g)

    def _wait_g(bufs):
        pltpu.make_async_copy(rec_h.at[bufs[0]], bufs[3], sem_g).wait()
        pltpu.make_async_copy(ald2_h.at[bufs[1]], bufs[4], sem_g).wait()

    def _issue_sc(bufs):
        pltpu.async_copy(bufs[5], acc_sp.at[bufs[7]], sem_s, add=True)

        @pl.when(cid == 0)
        def _():
            pltpu.async_copy(bufs[6], s_sp.at[bufs[7]], sem_s, add=True)

    def _wait_sc(bufs):
        pltpu.make_async_copy(bufs[5], acc_sp.at[bufs[7]], sem_s).wait()

        @pl.when(cid == 0)
        def _():
            pltpu.make_async_copy(bufs[6], s_sp.at[bufs[7]], sem_s).wait()

    def _vec(bufs):
        (_, dstr, ewr, recr, aldr, mr, pr, dlr) = bufs

        def vec(g, _):
            sl = pl.ds(g * L, L)
            ridx = iota + g * L
            hv = plsc.load_gather(recr, [ridx, cidv])
            av = plsc.load_gather(recr, [ridx, col2])
            e = av + aldr[sl]
            e = jnp.maximum(e, 0.2 * e)
            p = jnp.exp(e)
            mr[sl] = p * ewr[sl] * hv
            pr[sl] = p
            dlr[sl] = dstr[sl]
            return ()

        lax.fori_loop(0, K // L, vec, (), unroll=4)

    _issue_eb(A, 0)

    def body2(i, _):
        a = 2 * i
        b = a + 1
        _wait_eb(A, a)
        _issue_g(A)

        @pl.when(i > 0)
        def _():
            _wait_g(B)
            _vec(B)
            _issue_eb(B, b)
            _wait_sc(A)
            _issue_sc(B)

        @pl.when(i == 0)
        def _():
            _issue_eb(B, b)

        _wait_eb(B, b)
        _issue_g(B)
        _wait_g(A)
        _vec(A)

        @pl.when(i < CHUNKS // 2 - 1)
        def _():
            _issue_eb(A, a + 2)

        @pl.when(i > 0)
        def _():
            _wait_sc(B)
        _issue_sc(A)
        return ()

    lax.fori_loop(0, CHUNKS // 2, body2, ())
    _wait_g(B)
    _vec(B)
    _wait_sc(A)
    _issue_sc(B)
    _wait_sc(B)

    plsc.subcore_barrier()

    @pl.when(sid < 10)
    def _export():
        for t in range(STAGE // SB):
            rs = pl.ds(sid * STAGE + t * SB, SB)

            @pl.when(cid == 0)
            def _():
                pltpu.sync_copy(acc_sp.at[rs], sb)
                pltpu.sync_copy(sb, acc0_o.at[rs])
                pltpu.sync_copy(s_sp.at[rs], sb)
                pltpu.sync_copy(sb, s_o.at[rs])

            @pl.when(cid == 1)
            def _():
                pltpu.sync_copy(acc_sp.at[rs], sb)
                pltpu.sync_copy(sb, acc1_o.at[rs])


def _k2_bufs():
    return [
        pltpu.VMEM((K,), jnp.int32),
        pltpu.VMEM((K,), jnp.int32),
        pltpu.VMEM((K,), jnp.float32),
        pltpu.VMEM((K, 16), jnp.float32),
        pltpu.VMEM((K,), jnp.float32),
        pltpu.VMEM((K,), jnp.float32),
        pltpu.VMEM((K,), jnp.float32),
        pltpu.VMEM((K,), jnp.int32),
    ]


_layer2_call = pl.kernel(
    _sc_layer2,
    out_type=(
        jax.ShapeDtypeStruct((N,), jnp.float32),
        jax.ShapeDtypeStruct((N,), jnp.float32),
        jax.ShapeDtypeStruct((N,), jnp.float32),
    ),
    mesh=_mesh,
    compiler_params=_SC_PARAMS,
    scratch_types=[
        pltpu.VMEM_SHARED((N,), jnp.float32),
        pltpu.VMEM_SHARED((N,), jnp.float32),
        *_k2_bufs(),
        *_k2_bufs(),
        pltpu.VMEM((SB,), jnp.float32),
        pltpu.SemaphoreType.DMA,
        pltpu.SemaphoreType.DMA,
        pltpu.SemaphoreType.DMA,
    ],
)


_BR = 4000
_DOT = functools.partial(lax.dot_general,
                         dimension_numbers=(((1,), (0,)), ((), ())),
                         preferred_element_type=jnp.float32)


def _tc_proj_body(xu_ref, xb_ref, w1s_ref, w1d_ref, a1s_ref, a1d_ref, w2s_ref,
                  a2s_ref, h_ref, als_ref, ald_ref, rec_ref):
    h1 = _DOT(xu_ref[...], w1s_ref[...])
    hd1 = _DOT(xb_ref[...], w1d_ref[...])
    h_ref[...] = h1
    als_ref[...] = jnp.sum(h1 * a1s_ref[...][None, :], axis=-1, keepdims=True)
    ald_ref[...] = jnp.sum(hd1 * a1d_ref[...][None, :], axis=-1, keepdims=True)
    h2 = _DOT(jnp.maximum(h1, 0.0), w2s_ref[...])
    als2 = jnp.sum(h2 * a2s_ref[...][None, :], axis=-1, keepdims=True)
    pad = jnp.zeros((h2.shape[0], 13), jnp.float32)
    rec_ref[...] = jnp.concatenate([h2, als2, pad], axis=-1)


def _tc_mid_body(acc_ref, s_ref, b1_ref, w2d_ref, a2d_ref, ald2_ref):
    badge = jnp.maximum(
        acc_ref[...] / (s_ref[...] + EPS) + b1_ref[...][None, :], 0.0)
    hd2 = _DOT(badge, w2d_ref[...])
    ald2_ref[...] = jnp.sum(hd2 * a2d_ref[...][None, :], axis=-1, keepdims=True)


def _tc_out_body(a0_ref, a1_ref, s_ref, b2_ref, o_ref):
    acc = jnp.concatenate([a0_ref[...], a1_ref[...]], axis=-1)
    o_ref[...] = acc / (s_ref[...] + EPS) + b2_ref[...][None, :]


def kernel(x_user, x_badge, edge_index, edge_weight,
           W1_src, W1_dst, att1_src, att1_dst, b1,
           W2_src, W2_dst, att2_src, att2_dst, b2):
    src = edge_index[0]
    dst = edge_index[1]

    grid = (N // _BR,)
    full = lambda shp: pl.BlockSpec(shp, lambda i: tuple(0 for _ in shp))
    row2 = lambda w: pl.BlockSpec((_BR, w), lambda i: (i, 0))

    h1, als1, ald1, rec2 = pl.pallas_call(
        _tc_proj_body,
        grid=grid,
        in_specs=[row2(128), row2(128), full((128, 16)), full((128, 16)),
                  full((16,)), full((16,)), full((16, 2)), full((2,))],
        out_specs=[row2(16), row2(1), row2(1), row2(16)],
        out_shape=[
            jax.ShapeDtypeStruct((N, 16), jnp.float32),
            jax.ShapeDtypeStruct((N, 1), jnp.float32),
            jax.ShapeDtypeStruct((N, 1), jnp.float32),
            jax.ShapeDtypeStruct((N, 16), jnp.float32),
        ],
    )(x_user, x_badge, W1_src, W1_dst, att1_src, att1_dst, W2_src, att2_src)

    acc1, s1 = _layer1_call(
        src, dst, edge_weight, h1,
        als1.reshape(N), ald1.reshape(N))

    ald2 = pl.pallas_call(
        _tc_mid_body,
        grid=grid,
        in_specs=[row2(16), row2(1), full((16,)), full((16, 2)), full((2,))],
        out_specs=row2(1),
        out_shape=jax.ShapeDtypeStruct((N, 1), jnp.float32),
    )(acc1, s1.reshape(N, 1), b1, W2_dst, att2_dst)

    acc20, acc21, s2 = _layer2_call(
        src, dst, edge_weight, rec2, ald2.reshape(N))

    out = pl.pallas_call(
        _tc_out_body,
        grid=grid,
        in_specs=[row2(1), row2(1), row2(1), full((2,))],
        out_specs=row2(2),
        out_shape=jax.ShapeDtypeStruct((N, 2), jnp.float32),
    )(acc20.reshape(N, 1), acc21.reshape(N, 1), s2.reshape(N, 1), b2)
    return out


# speedup vs baseline: 68.9490x; 1.0556x over previous
import functools

import jax
import jax.numpy as jnp
from jax import lax
from jax.experimental import pallas as pl
from jax.experimental.pallas import tpu as pltpu
from jax.experimental.pallas import tpu_sc as plsc

N = 100000
NH = N // 2
E = 1600000
NS = 16
L = 16

K = 400
SB = 2000
EPS = 1e-16
CHUNKS = E // NS // K
STAGE = 10000
MIDR = 5000

_mesh = plsc.VectorSubcoreMesh(core_axis_name="c", subcore_axis_name="s")
_SC_PARAMS = pltpu.CompilerParams(needs_layout_passes=False,
                                  use_tc_tiling_on_sc=False)


def _sc_fused(src_h, dst_h, ew_h, h_hbm, als_h, ald_h, rec_h, b1_h, wt_h,
              acc20_o, acc21_o, s2_o, ald2_o,
              acc_sp, s_sp, a20_sp, a21_sp, s2_sp,
              src_a, dst_a, ew_a, als_a, ald_a, hrow_a, msg_a, p_a, dl_a,
              src_b, dst_b, ew_b, als_b, ald_b, hrow_b, msg_b, p_b, dl_b,
              m0_a, m1_a, m0_b, m1_b, c_b, sb, b1_v, wt_v,
              sem_e, sem_g, sem_s):
    cid = lax.axis_index("c")
    sid = lax.axis_index("s")
    is_hi = cid == 1

    A = (src_a, dst_a, ew_a, als_a, ald_a, hrow_a, msg_a, p_a, dl_a,
         m0_a, m1_a)
    B = (src_b, dst_b, ew_b, als_b, ald_b, hrow_b, msg_b, p_b, dl_b,
         m0_b, m1_b)

    zv = jnp.zeros((L,), jnp.float32)

    def _zm(j, _):
        msg_a[j] = zv
        return ()

    lax.fori_loop(0, K, _zm, (), unroll=8)

    def _zs(j, _):
        sb[pl.ds(j * L, L)] = zv
        return ()

    lax.fori_loop(0, SB // L, _zs, (), unroll=8)

    @pl.when(sid < 10)
    def _zero_acc():
        for t in range(MIDR // 200):
            rs = pl.ds(sid * MIDR + t * 200, 200)
            pltpu.sync_copy(msg_a.at[pl.ds(0, 200)], acc_sp.at[rs])

    @pl.when(sid < 5)
    def _zero_s():
        for t in range(STAGE // SB):
            rs = pl.ds(sid * STAGE + t * SB, SB)
            pltpu.sync_copy(sb, s_sp.at[rs])
            pltpu.sync_copy(sb, a20_sp.at[rs])
            pltpu.sync_copy(sb, a21_sp.at[rs])
            pltpu.sync_copy(sb, s2_sp.at[rs])

    pltpu.sync_copy(b1_h, b1_v)
    pltpu.sync_copy(wt_h, wt_v)

    plsc.subcore_barrier()

    ebase = sid * (E // NS)

    def _issue_eb(bufs, k):
        es = pl.ds(ebase + k * K, K)
        pltpu.async_copy(src_h.at[es], bufs[0], sem_e)
        pltpu.async_copy(dst_h.at[es], bufs[1], sem_e)
        pltpu.async_copy(ew_h.at[es], bufs[2], sem_e)

    def _wait_eb(bufs, k):
        es = pl.ds(ebase + k * K, K)
        pltpu.make_async_copy(src_h.at[es], bufs[0], sem_e).wait()
        pltpu.make_async_copy(dst_h.at[es], bufs[1], sem_e).wait()
        pltpu.make_async_copy(ew_h.at[es], bufs[2], sem_e).wait()

    def _issue_g1(bufs):
        pltpu.async_copy(als_h.at[bufs[0]], bufs[3], sem_g)
        pltpu.async_copy(ald_h.at[bufs[1]], bufs[4], sem_g)
        pltpu.async_copy(h_hbm.at[bufs[0]], bufs[5], sem_g)

    def _wait_g1(bufs):
        pltpu.make_async_copy(als_h.at[bufs[0]], bufs[3], sem_g).wait()
        pltpu.make_async_copy(ald_h.at[bufs[1]], bufs[4], sem_g).wait()
        pltpu.make_async_copy(h_hbm.at[bufs[0]], bufs[5], sem_g).wait()

    def _issue_sc1(bufs):
        pltpu.async_copy(bufs[6], acc_sp.at[bufs[8]], sem_s, add=True)
        pltpu.async_copy(bufs[7], s_sp.at[bufs[8]], sem_s, add=True)

    def _wait_sc1(bufs):
        pltpu.make_async_copy(bufs[6], acc_sp.at[bufs[8]], sem_s).wait()
        pltpu.make_async_copy(bufs[7], s_sp.at[bufs[8]], sem_s).wait()

    def _vec1(bufs):
        (_, dstr, ewr, alsr, aldr, _, _, pr, dlr, _, _) = bufs

        def vec(g, _):
            sl = pl.ds(g * L, L)
            e = alsr[sl] + aldr[sl]
            e = jnp.maximum(e, 0.2 * e)
            p = jnp.exp(e)
            dv = dstr[sl]
            ge = dv >= NH
            mine = ge == is_hi
            zero = jnp.zeros((L,), jnp.float32)
            c_b[sl] = jnp.where(mine, p * ewr[sl], zero)
            pr[sl] = jnp.where(mine, p, zero)
            dlr[sl] = jnp.where(ge, dv - NH, dv)
            return ()

        lax.fori_loop(0, K // L, vec, (), unroll=4)

    def _mrow1(bufs):
        hrowr = bufs[5]
        msgr = bufs[6]

        def mrow(g, _):
            base = g * L
            c16 = c_b[pl.ds(base, L)]
            for l in range(L):
                msgr[base + l] = hrowr[base + l] * c16[l]
            return ()

        lax.fori_loop(0, K // L, mrow, (), unroll=1)

    def _run_pipeline(issue_g, wait_g, vec, mrow, issue_sc, wait_sc):
        _issue_eb(A, 0)

        def body2(i, _):
            a = 2 * i
            b = a + 1
            _wait_eb(A, a)
            issue_g(A)

            @pl.when(i > 0)
            def _():
                wait_g(B)
                vec(B)
                _issue_eb(B, b)
                mrow(B)
                wait_sc(A)
                issue_sc(B)

            @pl.when(i == 0)
            def _():
                _issue_eb(B, b)

            _wait_eb(B, b)
            issue_g(B)
            wait_g(A)
            vec(A)

            @pl.when(i < CHUNKS // 2 - 1)
            def _():
                _issue_eb(A, a + 2)

            mrow(A)

            @pl.when(i > 0)
            def _():
                wait_sc(B)
            issue_sc(A)
            return ()

        lax.fori_loop(0, CHUNKS // 2, body2, ())
        wait_g(B)
        vec(B)
        mrow(B)
        wait_sc(A)
        issue_sc(B)
        wait_sc(B)

    _run_pipeline(_issue_g1, _wait_g1, _vec1, _mrow1, _issue_sc1, _wait_sc1)

    plsc.subcore_barrier()

    iota = lax.iota(jnp.int32, L)
    b1_vec = b1_v[...]
    wt_vec = wt_v[...]

    @pl.when(sid < 10)
    def _mid():
        for t in range(MIDR // 200):
            loc = sid * MIDR + t * 200
            pltpu.sync_copy(acc_sp.at[pl.ds(loc, 200)],
                            msg_a.at[pl.ds(0, 200)])
            pltpu.sync_copy(s_sp.at[pl.ds(loc, 200)], p_a.at[pl.ds(0, 200)])

            def mg(g, _):
                sl = pl.ds(g * L, L)
                ridx = iota + g * L
                inv = 1.0 / (p_a[sl] + EPS)
                acc0 = jnp.zeros((L,), jnp.float32)
                for f in range(16):
                    colv = plsc.load_gather(
                        msg_a, [ridx, jnp.full((L,), f, jnp.int32)])
                    acc0 = acc0 + jnp.maximum(
                        colv * inv + b1_vec[f], 0.0) * wt_vec[f]
                als_a[sl] = acc0
                return ()

            lax.fori_loop(0, 13, mg, ())
            pltpu.sync_copy(als_a.at[pl.ds(0, 200)],
                            ald2_o.at[pl.ds(cid * NH + loc, 200)])

    plsc.subcore_barrier()

    c0v = jnp.zeros((L,), jnp.int32)
    c1v = jnp.full((L,), 1, jnp.int32)
    c2v = jnp.full((L,), 2, jnp.int32)

    def _issue_g2(bufs):
        pltpu.async_copy(rec_h.at[bufs[0]], bufs[5], sem_g)
        pltpu.async_copy(ald2_o.at[bufs[1]], bufs[4], sem_g)

    def _wait_g2(bufs):
        pltpu.make_async_copy(rec_h.at[bufs[0]], bufs[5], sem_g).wait()
        pltpu.make_async_copy(ald2_o.at[bufs[1]], bufs[4], sem_g).wait()

    def _issue_sc2(bufs):
        pltpu.async_copy(bufs[9], a20_sp.at[bufs[8]], sem_s, add=True)
        pltpu.async_copy(bufs[10], a21_sp.at[bufs[8]], sem_s, add=True)
        pltpu.async_copy(bufs[7], s2_sp.at[bufs[8]], sem_s, add=True)

    def _wait_sc2(bufs):
        pltpu.make_async_copy(bufs[9], a20_sp.at[bufs[8]], sem_s).wait()
        pltpu.make_async_copy(bufs[10], a21_sp.at[bufs[8]], sem_s).wait()
        pltpu.make_async_copy(bufs[7], s2_sp.at[bufs[8]], sem_s).wait()

    def _vec2(bufs):
        (_, dstr, ewr, _, aldr, hrowr, _, pr, dlr, m0r, m1r) = bufs

        def vec(g, _):
            sl = pl.ds(g * L, L)
            ridx = iota + g * L
            hv0 = plsc.load_gather(hrowr, [ridx, c0v])
            hv1 = plsc.load_gather(hrowr, [ridx, c1v])
            av = plsc.load_gather(hrowr, [ridx, c2v])
            e = av + aldr[sl]
            e = jnp.maximum(e, 0.2 * e)
            p = jnp.exp(e)
            dv = dstr[sl]
            ge = dv >= NH
            mine = ge == is_hi
            zero = jnp.zeros((L,), jnp.float32)
            c = jnp.where(mine, p * ewr[sl], zero)
            pr[sl] = jnp.where(mine, p, zero)
            m0r[sl] = c * hv0
            m1r[sl] = c * hv1
            dlr[sl] = jnp.where(ge, dv - NH, dv)
            return ()

        lax.fori_loop(0, K // L, vec, (), unroll=4)

    def _mrow2(bufs):
        pass

    _run_pipeline(_issue_g2, _wait_g2, _vec2, _mrow2, _issue_sc2, _wait_sc2)

    plsc.subcore_barrier()

    @pl.when(sid < 5)
    def _export():
        for t in range(STAGE // SB):
            loc = sid * STAGE + t * SB
            rs = pl.ds(loc, SB)
            rg = pl.ds(cid * NH + loc, SB)
            pltpu.sync_copy(a20_sp.at[rs], sb)
            pltpu.sync_copy(sb, acc20_o.at[rg])
            pltpu.sync_copy(a21_sp.at[rs], sb)
            pltpu.sync_copy(sb, acc21_o.at[rg])
            pltpu.sync_copy(s2_sp.at[rs], sb)
            pltpu.sync_copy(sb, s2_o.at[rg])


def _k_bufs():
    return [
        pltpu.VMEM((K,), jnp.int32),
        pltpu.VMEM((K,), jnp.int32),
        pltpu.VMEM((K,), jnp.float32),
        pltpu.VMEM((K,), jnp.float32),
        pltpu.VMEM((K,), jnp.float32),
        pltpu.VMEM((K, 16), jnp.float32),
        pltpu.VMEM((K, 16), jnp.float32),
        pltpu.VMEM((K,), jnp.float32),
        pltpu.VMEM((K,), jnp.int32),
    ]


_fused_call = pl.kernel(
    _sc_fused,
    out_type=(
        jax.ShapeDtypeStruct((N,), jnp.float32),
        jax.ShapeDtypeStruct((N,), jnp.float32),
        jax.ShapeDtypeStruct((N,), jnp.float32),
        jax.ShapeDtypeStruct((N,), jnp.float32),
    ),
    mesh=_mesh,
    compiler_params=_SC_PARAMS,
    scratch_types=[
        pltpu.VMEM_SHARED((NH, 16), jnp.float32),
        pltpu.VMEM_SHARED((NH,), jnp.float32),
        pltpu.VMEM_SHARED((NH,), jnp.float32),
        pltpu.VMEM_SHARED((NH,), jnp.float32),
        pltpu.VMEM_SHARED((NH,), jnp.float32),
        *_k_bufs(),
        *_k_bufs(),
        pltpu.VMEM((K,), jnp.float32),
        pltpu.VMEM((K,), jnp.float32),
        pltpu.VMEM((K,), jnp.float32),
        pltpu.VMEM((K,), jnp.float32),
        pltpu.VMEM((K,), jnp.float32),
        pltpu.VMEM((SB,), jnp.float32),
        pltpu.VMEM((16,), jnp.float32),
        pltpu.VMEM((16,), jnp.float32),
        pltpu.SemaphoreType.DMA,
        pltpu.SemaphoreType.DMA,
        pltpu.SemaphoreType.DMA,
    ],
)


_BR = 4000
_DOT = functools.partial(lax.dot_general,
                         dimension_numbers=(((1,), (0,)), ((), ())),
                         preferred_element_type=jnp.float32)


def _tc_proj_body(xu_ref, xb_ref, w1s_ref, w1d_ref, a1s_ref, a1d_ref, w2s_ref,
                  a2s_ref, w2d_ref, a2d_ref,
                  h_ref, als_ref, ald_ref, rec_ref, wt_ref):
    h1 = _DOT(xu_ref[...], w1s_ref[...])
    hd1 = _DOT(xb_ref[...], w1d_ref[...])
    h_ref[...] = h1
    als_ref[...] = jnp.sum(h1 * a1s_ref[...][None, :], axis=-1, keepdims=True)
    ald_ref[...] = jnp.sum(hd1 * a1d_ref[...][None, :], axis=-1, keepdims=True)
    h2 = _DOT(jnp.maximum(h1, 0.0), w2s_ref[...])
    als2 = jnp.sum(h2 * a2s_ref[...][None, :], axis=-1, keepdims=True)
    pad = jnp.zeros((h2.shape[0], 13), jnp.float32)
    rec_ref[...] = jnp.concatenate([h2, als2, pad], axis=-1)
    wt_ref[...] = jnp.sum(w2d_ref[...] * a2d_ref[...][None, :],
                          axis=-1).reshape(1, 16)


def _tc_out_body(a0_ref, a1_ref, s_ref, b2_ref, o_ref):
    acc = jnp.concatenate([a0_ref[...], a1_ref[...]], axis=-1)
    o_ref[...] = acc / (s_ref[...] + EPS) + b2_ref[...][None, :]


def kernel(x_user, x_badge, edge_index, edge_weight,
           W1_src, W1_dst, att1_src, att1_dst, b1,
           W2_src, W2_dst, att2_src, att2_dst, b2):
    src = edge_index[0]
    dst = edge_index[1]

    grid = (N // _BR,)
    full = lambda shp: pl.BlockSpec(shp, lambda i: tuple(0 for _ in shp))
    row2 = lambda w: pl.BlockSpec((_BR, w), lambda i: (i, 0))

    h1, als1, ald1, rec2, wt = pl.pallas_call(
        _tc_proj_body,
        grid=grid,
        in_specs=[row2(128), row2(128), full((128, 16)), full((128, 16)),
                  full((16,)), full((16,)), full((16, 2)), full((2,)),
                  full((16, 2)), full((2,))],
        out_specs=[row2(16), row2(1), row2(1), row2(16), full((1, 16))],
        out_shape=[
            jax.ShapeDtypeStruct((N, 16), jnp.float32),
            jax.ShapeDtypeStruct((N, 1), jnp.float32),
            jax.ShapeDtypeStruct((N, 1), jnp.float32),
            jax.ShapeDtypeStruct((N, 16), jnp.float32),
            jax.ShapeDtypeStruct((1, 16), jnp.float32),
        ],
    )(x_user, x_badge, W1_src, W1_dst, att1_src, att1_dst, W2_src, att2_src,
      W2_dst, att2_dst)

    acc20, acc21, s2, _ald2 = _fused_call(
        src, dst, edge_weight, h1,
        als1.reshape(N), ald1.reshape(N), rec2, b1, wt.reshape(16))

    out = pl.pallas_call(
        _tc_out_body,
        grid=grid,
        in_specs=[row2(1), row2(1), row2(1), full((2,))],
        out_specs=row2(2),
        out_shape=jax.ShapeDtypeStruct((N, 2), jnp.float32),
    )(acc20.reshape(N, 1), acc21.reshape(N, 1), s2.reshape(N, 1), b2)
    return out


# speedup vs baseline: 73.6501x; 1.0682x over previous
import functools

import jax
import jax.numpy as jnp
from jax import lax
from jax.experimental import pallas as pl
from jax.experimental.pallas import tpu as pltpu
from jax.experimental.pallas import tpu_sc as plsc

N = 100000
NH = N // 2
E = 1600000
NS = 16
L = 16

K = 400
SB = 2000
EPS = 1e-16
CHUNKS = E // NS // K
STAGE = 10000
MIDR = 5000

_mesh = plsc.VectorSubcoreMesh(core_axis_name="c", subcore_axis_name="s")
_SC_PARAMS = pltpu.CompilerParams(needs_layout_passes=False,
                                  use_tc_tiling_on_sc=False)


def _sc_fused(src_h, dst_h, ew_h, h_hbm, als_h, ald_h, rec_h, b1_h, wt_h,
              acc2i_o, ald2_o,
              acc_sp, s_sp, acc2_sp,
              src_a, dst_a, ew_a, als_a, ald_a, hrow_a, msg_a, p_a, dl_a,
              src_b, dst_b, ew_b, als_b, ald_b, hrow_b, msg_b, p_b, dl_b,
              msg2_a, msg2_b, c_b, sb, b1_v, wt_v,
              sem_e, sem_g, sem_s):
    cid = lax.axis_index("c")
    sid = lax.axis_index("s")
    is_hi = cid == 1

    A = (src_a, dst_a, ew_a, als_a, ald_a, hrow_a, msg_a, p_a, dl_a, msg2_a)
    B = (src_b, dst_b, ew_b, als_b, ald_b, hrow_b, msg_b, p_b, dl_b, msg2_b)

    zv = jnp.zeros((L,), jnp.float32)

    def _zm(j, _):
        msg_a[j] = zv
        return ()

    lax.fori_loop(0, K, _zm, (), unroll=8)

    def _zs(j, _):
        sb[pl.ds(j * L, L)] = zv
        return ()

    lax.fori_loop(0, SB // L, _zs, (), unroll=8)

    @pl.when(sid < 10)
    def _zero_acc():
        for t in range(MIDR // 200):
            rs = pl.ds(sid * MIDR + t * 200, 200)
            pltpu.sync_copy(msg_a.at[pl.ds(0, 200)], acc_sp.at[rs])

    @pl.when(sid < 5)
    def _zero_s():
        for t in range(STAGE // SB):
            rs = pl.ds(sid * STAGE + t * SB, SB)
            pltpu.sync_copy(sb, s_sp.at[rs])

    riota = lax.iota(jnp.int32, L) >> 3
    ciota = lax.iota(jnp.int32, L) & 7
    zv16 = jnp.zeros((L,), jnp.float32)

    def _zm2(j, _):
        rr = riota + 2 * j
        plsc.store_scatter(msg2_a, [rr, ciota], zv16)
        plsc.store_scatter(msg2_b, [rr, ciota], zv16)
        return ()

    lax.fori_loop(0, K // 2, _zm2, (), unroll=8)

    @pl.when(sid < 10)
    def _zero_acc2():
        for t in range(MIDR // 200):
            rs = pl.ds(sid * MIDR + t * 200, 200)
            pltpu.sync_copy(msg2_a.at[pl.ds(0, 200)], acc2_sp.at[rs])

    pltpu.sync_copy(b1_h, b1_v)
    pltpu.sync_copy(wt_h, wt_v)

    plsc.subcore_barrier()

    ebase = sid * (E // NS)

    def _issue_eb(bufs, k):
        es = pl.ds(ebase + k * K, K)
        pltpu.async_copy(src_h.at[es], bufs[0], sem_e)
        pltpu.async_copy(dst_h.at[es], bufs[1], sem_e)
        pltpu.async_copy(ew_h.at[es], bufs[2], sem_e)

    def _wait_eb(bufs, k):
        es = pl.ds(ebase + k * K, K)
        pltpu.make_async_copy(src_h.at[es], bufs[0], sem_e).wait()
        pltpu.make_async_copy(dst_h.at[es], bufs[1], sem_e).wait()
        pltpu.make_async_copy(ew_h.at[es], bufs[2], sem_e).wait()

    def _issue_g1(bufs):
        pltpu.async_copy(als_h.at[bufs[0]], bufs[3], sem_g)
        pltpu.async_copy(ald_h.at[bufs[1]], bufs[4], sem_g)
        pltpu.async_copy(h_hbm.at[bufs[0]], bufs[5], sem_g)

    def _wait_g1(bufs):
        pltpu.make_async_copy(als_h.at[bufs[0]], bufs[3], sem_g).wait()
        pltpu.make_async_copy(ald_h.at[bufs[1]], bufs[4], sem_g).wait()
        pltpu.make_async_copy(h_hbm.at[bufs[0]], bufs[5], sem_g).wait()

    def _issue_sc1(bufs):
        pltpu.async_copy(bufs[6], acc_sp.at[bufs[8]], sem_s, add=True)
        pltpu.async_copy(bufs[7], s_sp.at[bufs[8]], sem_s, add=True)

    def _wait_sc1(bufs):
        pltpu.make_async_copy(bufs[6], acc_sp.at[bufs[8]], sem_s).wait()
        pltpu.make_async_copy(bufs[7], s_sp.at[bufs[8]], sem_s).wait()

    def _vec1(bufs):
        (_, dstr, ewr, alsr, aldr, _, _, pr, dlr, _) = bufs

        def vec(g, _):
            sl = pl.ds(g * L, L)
            e = alsr[sl] + aldr[sl]
            e = jnp.maximum(e, 0.2 * e)
            p = jnp.exp(e)
            dv = dstr[sl]
            ge = dv >= NH
            mine = ge == is_hi
            zero = jnp.zeros((L,), jnp.float32)
            c_b[sl] = jnp.where(mine, p * ewr[sl], zero)
            pr[sl] = jnp.where(mine, p, zero)
            dlr[sl] = jnp.where(ge, dv - NH, dv)
            return ()

        lax.fori_loop(0, K // L, vec, (), unroll=4)

    def _mrow1(bufs):
        hrowr = bufs[5]
        msgr = bufs[6]

        def mrow(g, _):
            base = g * L
            c16 = c_b[pl.ds(base, L)]
            for l in range(L):
                msgr[base + l] = hrowr[base + l] * c16[l]
            return ()

        lax.fori_loop(0, K // L, mrow, (), unroll=1)

    def _run_pipeline(issue_g, wait_g, vec, mrow, issue_sc, wait_sc):
        _issue_eb(A, 0)

        def body2(i, _):
            a = 2 * i
            b = a + 1
            _wait_eb(A, a)
            issue_g(A)

            @pl.when(i > 0)
            def _():
                wait_g(B)
                vec(B)
                _issue_eb(B, b)
                mrow(B)
                wait_sc(A)
                issue_sc(B)

            @pl.when(i == 0)
            def _():
                _issue_eb(B, b)

            _wait_eb(B, b)
            issue_g(B)
            wait_g(A)
            vec(A)

            @pl.when(i < CHUNKS // 2 - 1)
            def _():
                _issue_eb(A, a + 2)

            mrow(A)

            @pl.when(i > 0)
            def _():
                wait_sc(B)
            issue_sc(A)
            return ()

        lax.fori_loop(0, CHUNKS // 2, body2, ())
        wait_g(B)
        vec(B)
        mrow(B)
        wait_sc(A)
        issue_sc(B)
        wait_sc(B)

    _run_pipeline(_issue_g1, _wait_g1, _vec1, _mrow1, _issue_sc1, _wait_sc1)

    plsc.subcore_barrier()

    iota = lax.iota(jnp.int32, L)
    b1_vec = b1_v[...]
    wt_vec = wt_v[...]

    @pl.when(sid < 10)
    def _mid():
        for t in range(MIDR // 200):
            loc = sid * MIDR + t * 200
            pltpu.sync_copy(acc_sp.at[pl.ds(loc, 200)],
                            msg_a.at[pl.ds(0, 200)])
            pltpu.sync_copy(s_sp.at[pl.ds(loc, 200)], p_a.at[pl.ds(0, 200)])

            def mg(g, _):
                sl = pl.ds(g * L, L)
                ridx = iota + g * L
                inv = 1.0 / (p_a[sl] + EPS)
                acc0 = jnp.zeros((L,), jnp.float32)
                for f in range(16):
                    colv = plsc.load_gather(
                        msg_a, [ridx, jnp.full((L,), f, jnp.int32)])
                    acc0 = acc0 + jnp.maximum(
                        colv * inv + b1_vec[f], 0.0) * wt_vec[f]
                als_a[sl] = acc0
                return ()

            lax.fori_loop(0, 13, mg, ())
            pltpu.sync_copy(als_a.at[pl.ds(0, 200)],
                            ald2_o.at[pl.ds(cid * NH + loc, 200)])

    plsc.subcore_barrier()

    c0v = jnp.zeros((L,), jnp.int32)
    c1v = jnp.full((L,), 1, jnp.int32)
    c2v = jnp.full((L,), 2, jnp.int32)

    def _issue_g2(bufs):
        pltpu.async_copy(rec_h.at[bufs[0]], bufs[5], sem_g)
        pltpu.async_copy(ald2_o.at[bufs[1]], bufs[4], sem_g)

    def _wait_g2(bufs):
        pltpu.make_async_copy(rec_h.at[bufs[0]], bufs[5], sem_g).wait()
        pltpu.make_async_copy(ald2_o.at[bufs[1]], bufs[4], sem_g).wait()

    def _issue_sc2(bufs):
        pltpu.async_copy(bufs[9], acc2_sp.at[bufs[8]], sem_s, add=True)

    def _wait_sc2(bufs):
        pltpu.make_async_copy(bufs[9], acc2_sp.at[bufs[8]], sem_s).wait()

    def _vec2(bufs):
        (_, dstr, ewr, _, aldr, hrowr, _, _, dlr, m2r) = bufs

        def vec(g, _):
            sl = pl.ds(g * L, L)
            ridx = iota + g * L
            hv0 = plsc.load_gather(hrowr, [ridx, c0v])
            hv1 = plsc.load_gather(hrowr, [ridx, c1v])
            av = plsc.load_gather(hrowr, [ridx, c2v])
            e = av + aldr[sl]
            e = jnp.maximum(e, 0.2 * e)
            p = jnp.exp(e)
            dv = dstr[sl]
            ge = dv >= NH
            mine = ge == is_hi
            zero = jnp.zeros((L,), jnp.float32)
            c = jnp.where(mine, p * ewr[sl], zero)
            pm = jnp.where(mine, p, zero)
            plsc.store_scatter(m2r, [ridx, c0v], c * hv0)
            plsc.store_scatter(m2r, [ridx, c1v], c * hv1)
            plsc.store_scatter(m2r, [ridx, c2v], pm)
            dlr[sl] = jnp.where(ge, dv - NH, dv)
            return ()

        lax.fori_loop(0, K // L, vec, (), unroll=4)

    def _mrow2(bufs):
        pass

    _run_pipeline(_issue_g2, _wait_g2, _vec2, _mrow2, _issue_sc2, _wait_sc2)

    plsc.subcore_barrier()

    @pl.when(sid < 10)
    def _export():
        for t in range(MIDR // 200):
            loc = sid * MIDR + t * 200
            rs = pl.ds(loc, 200)
            rg = pl.ds(cid * NH + loc, 200)
            pltpu.sync_copy(acc2_sp.at[rs], msg2_a.at[pl.ds(0, 200)])
            pltpu.sync_copy(msg2_a.at[pl.ds(0, 200)], acc2i_o.at[rg])


def _k_bufs():
    return [
        pltpu.VMEM((K,), jnp.int32),
        pltpu.VMEM((K,), jnp.int32),
        pltpu.VMEM((K,), jnp.float32),
        pltpu.VMEM((K,), jnp.float32),
        pltpu.VMEM((K,), jnp.float32),
        pltpu.VMEM((K, 16), jnp.float32),
        pltpu.VMEM((K, 16), jnp.float32),
        pltpu.VMEM((K,), jnp.float32),
        pltpu.VMEM((K,), jnp.int32),
    ]


_fused_call = pl.kernel(
    _sc_fused,
    out_type=(
        jax.ShapeDtypeStruct((N, 8), jnp.float32),
        jax.ShapeDtypeStruct((N,), jnp.float32),
    ),
    mesh=_mesh,
    compiler_params=_SC_PARAMS,
    scratch_types=[
        pltpu.VMEM_SHARED((NH, 16), jnp.float32),
        pltpu.VMEM_SHARED((NH,), jnp.float32),
        pltpu.VMEM_SHARED((NH, 8), jnp.float32),
        *_k_bufs(),
        *_k_bufs(),
        pltpu.VMEM((K, 8), jnp.float32),
        pltpu.VMEM((K, 8), jnp.float32),
        pltpu.VMEM((K,), jnp.float32),
        pltpu.VMEM((SB,), jnp.float32),
        pltpu.VMEM((16,), jnp.float32),
        pltpu.VMEM((16,), jnp.float32),
        pltpu.SemaphoreType.DMA,
        pltpu.SemaphoreType.DMA,
        pltpu.SemaphoreType.DMA,
    ],
)


_BR = 4000
_DOT = functools.partial(lax.dot_general,
                         dimension_numbers=(((1,), (0,)), ((), ())),
                         preferred_element_type=jnp.float32)


def _tc_proj_body(xu_ref, xb_ref, w1s_ref, w1d_ref, a1s_ref, a1d_ref, w2s_ref,
                  a2s_ref, w2d_ref, a2d_ref,
                  h_ref, als_ref, ald_ref, rec_ref, wt_ref):
    h1 = _DOT(xu_ref[...], w1s_ref[...])
    hd1 = _DOT(xb_ref[...], w1d_ref[...])
    h_ref[...] = h1
    als_ref[...] = jnp.sum(h1 * a1s_ref[...][None, :], axis=-1, keepdims=True)
    ald_ref[...] = jnp.sum(hd1 * a1d_ref[...][None, :], axis=-1, keepdims=True)
    h2 = _DOT(jnp.maximum(h1, 0.0), w2s_ref[...])
    als2 = jnp.sum(h2 * a2s_ref[...][None, :], axis=-1, keepdims=True)
    pad = jnp.zeros((h2.shape[0], 13), jnp.float32)
    rec_ref[...] = jnp.concatenate([h2, als2, pad], axis=-1)
    wt_ref[...] = jnp.sum(w2d_ref[...] * a2d_ref[...][None, :],
                          axis=-1).reshape(1, 16)


def _tc_out_body(ai_ref, b2_ref, o_ref):
    blk = ai_ref[...]
    acc = blk[:, 0:2]
    s = blk[:, 2:3]
    o_ref[...] = acc / (s + EPS) + b2_ref[...][None, :]


def kernel(x_user, x_badge, edge_index, edge_weight,
           W1_src, W1_dst, att1_src, att1_dst, b1,
           W2_src, W2_dst, att2_src, att2_dst, b2):
    src = edge_index[0]
    dst = edge_index[1]

    grid = (N // _BR,)
    full = lambda shp: pl.BlockSpec(shp, lambda i: tuple(0 for _ in shp))
    row2 = lambda w: pl.BlockSpec((_BR, w), lambda i: (i, 0))

    h1, als1, ald1, rec2, wt = pl.pallas_call(
        _tc_proj_body,
        grid=grid,
        in_specs=[row2(128), row2(128), full((128, 16)), full((128, 16)),
                  full((16,)), full((16,)), full((16, 2)), full((2,)),
                  full((16, 2)), full((2,))],
        out_specs=[row2(16), row2(1), row2(1), row2(16), full((1, 16))],
        out_shape=[
            jax.ShapeDtypeStruct((N, 16), jnp.float32),
            jax.ShapeDtypeStruct((N, 1), jnp.float32),
            jax.ShapeDtypeStruct((N, 1), jnp.float32),
            jax.ShapeDtypeStruct((N, 16), jnp.float32),
            jax.ShapeDtypeStruct((1, 16), jnp.float32),
        ],
    )(x_user, x_badge, W1_src, W1_dst, att1_src, att1_dst, W2_src, att2_src,
      W2_dst, att2_dst)

    acc2i, _ald2 = _fused_call(
        src, dst, edge_weight, h1,
        als1.reshape(N), ald1.reshape(N), rec2, b1, wt.reshape(16))

    out = pl.pallas_call(
        _tc_out_body,
        grid=grid,
        in_specs=[row2(8), full((2,))],
        out_specs=row2(2),
        out_shape=jax.ShapeDtypeStruct((N, 2), jnp.float32),
    )(acc2i, b2)
    return out
